# baseline (speedup 1.0000x reference)
import jax
import jax.numpy as jnp
from jax import lax
from jax.experimental import pallas as pl
from jax.experimental.pallas import tpu as pltpu
from jax.experimental.pallas import tpu_sc as plsc

ROWS = 8192
COLS = 512
LANES = 16
NSEG = 8
BROW = 16
NBLK = ROWS // BROW
BLKE = BROW * COLS
RBLK = 2048
BPG = RBLK // BROW


def _blocksum_body(x_ref, o_ref):
    r = lax.broadcasted_iota(jnp.int32, (BPG, RBLK), 0)
    c = lax.broadcasted_iota(jnp.int32, (BPG, RBLK), 1)
    sel = (lax.div(c, BROW) == r).astype(jnp.float32)
    part = jnp.dot(sel, x_ref[...], preferred_element_type=jnp.float32)
    o_ref[...] = jnp.sum(part, axis=1)


@jax.jit
def _blocksums(x):
    return pl.pallas_call(
        _blocksum_body,
        grid=(ROWS // RBLK,),
        in_specs=[pl.BlockSpec((RBLK, COLS), lambda i: (i, 0))],
        out_specs=pl.BlockSpec((BPG,), lambda i: (i,)),
        out_shape=jax.ShapeDtypeStruct((NBLK,), jnp.float32),
    )(x)


def _edges_body(x_hbm, s_hbm, o_hbm, sbuf, xbuf, accbuf):
    cid = lax.axis_index("c")
    sid = lax.axis_index("s")

    @pl.when(jnp.logical_and(cid == 0, sid < NSEG))
    def _():
        pltpu.sync_copy(s_hbm, sbuf)
        svec = sbuf[...]
        a = jnp.int32(0)
        b = jnp.int32(0)
        for i in range(NSEG):
            m = sid == i
            a = jnp.where(m, svec[i], a)
            b = jnp.where(m, svec[i + 1], b)

        accbuf[...] = jnp.zeros((LANES,), jnp.float32)

        @pl.when(b > a)
        def _():
            jb0 = lax.div(a, BROW)
            jb1 = lax.div(b - 1, BROW)

            def do_block(j):
                full = jnp.logical_and(j * BROW >= a,
                                       j * BROW + BROW <= b)

                @pl.when(jnp.logical_not(full))
                def _():
                    pltpu.sync_copy(x_hbm.at[pl.ds(j * BLKE, BLKE)], xbuf)
                    r0 = jnp.maximum(a - j * BROW, 0)
                    r1 = jnp.minimum(b - j * BROW, BROW)
                    vpr = COLS // LANES

                    def body(v, acc):
                        return acc + xbuf[pl.ds(v * LANES, LANES)]

                    acc = lax.fori_loop(r0 * vpr, r1 * vpr, body,
                                        jnp.zeros((LANES,), jnp.float32))
                    accbuf[...] = accbuf[...] + acc

            do_block(jb0)

            @pl.when(jb1 > jb0)
            def _():
                do_block(jb1)

        pltpu.sync_copy(accbuf, o_hbm.at[sid])


@jax.jit
def _edges(xf, s16):
    mesh = plsc.VectorSubcoreMesh(
        core_axis_name="c", subcore_axis_name="s", num_cores=2,
        num_subcores=16)
    f = pl.kernel(
        _edges_body,
        out_type=jax.ShapeDtypeStruct((NSEG, LANES), jnp.float32),
        mesh=mesh,
        scratch_types=[
            pltpu.VMEM((LANES,), jnp.int32),
            pltpu.VMEM((BLKE,), jnp.float32),
            pltpu.VMEM((LANES,), jnp.float32),
        ],
    )
    return f(xf, s16)


def _combine_body(q_ref, e_ref, s_ref, o_ref):
    q = q_ref[...]
    ii = (lax.broadcasted_iota(jnp.int32, (4, 128), 0) * 128
          + lax.broadcasted_iota(jnp.int32, (4, 128), 1))
    res = jnp.float32(1.0)
    for i in range(NSEG):
        a = s_ref[0, i]
        b = s_ref[0, i + 1]
        lo = lax.div(a + BROW - 1, BROW)
        hi = lax.div(b, BROW)
        m = jnp.logical_and(ii >= lo, ii < hi)
        bs = jnp.sum(jnp.where(m, q, 0.0))
        es = jnp.sum(e_ref[i, :])
        res = res * (bs + es)
    o_ref[...] = jnp.broadcast_to(res, (1, 1))


@jax.jit
def _combine(q, e, s16):
    return pl.pallas_call(
        _combine_body,
        out_shape=jax.ShapeDtypeStruct((1, 1), jnp.float32),
    )(q.reshape(4, 128), e, s16.reshape(1, LANES))


def kernel(x, slices):
    s16 = jnp.pad(slices.astype(jnp.int32), (0, 7))
    e = _edges(x.reshape(-1), s16)
    q = _blocksums(x)
    res = _combine(q, e, s16)
    return res[0, 0]


# speedup vs baseline: 1.2392x; 1.2392x over previous
import jax
import jax.numpy as jnp
from jax import lax
from jax.experimental import pallas as pl
from jax.experimental.pallas import tpu as pltpu
from jax.experimental.pallas import tpu_sc as plsc

ROWS = 8192
COLS = 512
LANES = 16
NSEG = 8
BROW = 16
NBLK = ROWS // BROW
BLKE = BROW * COLS
RBLK = 2048
BPG = RBLK // BROW


def _blocksum_body(x_ref, o_ref):
    r = lax.broadcasted_iota(jnp.int32, (BPG, RBLK), 0)
    c = lax.broadcasted_iota(jnp.int32, (BPG, RBLK), 1)
    sel = (lax.div(c, BROW) == r).astype(jnp.float32)
    part = jnp.dot(sel, x_ref[...], preferred_element_type=jnp.float32)
    o_ref[...] = jnp.sum(part, axis=1)


@jax.jit
def _blocksums(x):
    return pl.pallas_call(
        _blocksum_body,
        grid=(ROWS // RBLK,),
        in_specs=[pl.BlockSpec((RBLK, COLS), lambda i: (i, 0))],
        out_specs=pl.BlockSpec((BPG,), lambda i: (i,)),
        out_shape=jax.ShapeDtypeStruct((NBLK,), jnp.float32),
    )(x)


def _edges_body(x_hbm, s_hbm, o_hbm, sbuf, rowbuf, accbuf):
    cid = lax.axis_index("c")
    sid = lax.axis_index("s")

    @pl.when(jnp.logical_and(cid == 0, sid < NSEG))
    def _():
        pltpu.sync_copy(s_hbm, sbuf)
        svec = sbuf[...]
        a = jnp.int32(0)
        b = jnp.int32(0)
        for i in range(NSEG):
            m = sid == i
            a = jnp.where(m, svec[i], a)
            b = jnp.where(m, svec[i + 1], b)

        accbuf[...] = jnp.zeros((LANES,), jnp.float32)

        @pl.when(b > a)
        def _():
            jb0 = lax.div(a, BROW)
            jb1 = lax.div(b - 1, BROW)

            def do_block(j):
                full = jnp.logical_and(j * BROW >= a,
                                       j * BROW + BROW <= b)

                @pl.when(jnp.logical_not(full))
                def _():
                    r0 = jnp.maximum(a, j * BROW)
                    r1 = jnp.minimum(b, j * BROW + BROW)

                    def row_body(r, acc):
                        pltpu.sync_copy(x_hbm.at[r], rowbuf)

                        def col_body(c, acc2):
                            return acc2 + rowbuf[pl.ds(c * LANES, LANES)]

                        return lax.fori_loop(0, COLS // LANES, col_body,
                                             acc)

                    acc = lax.fori_loop(r0, r1, row_body,
                                        jnp.zeros((LANES,), jnp.float32))
                    accbuf[...] = accbuf[...] + acc

            do_block(jb0)

            @pl.when(jb1 > jb0)
            def _():
                do_block(jb1)

        pltpu.sync_copy(accbuf, o_hbm.at[sid])


@jax.jit
def _edges(xf, s16):
    mesh = plsc.VectorSubcoreMesh(
        core_axis_name="c", subcore_axis_name="s", num_cores=2,
        num_subcores=16)
    f = pl.kernel(
        _edges_body,
        out_type=jax.ShapeDtypeStruct((NSEG, LANES), jnp.float32),
        mesh=mesh,
        scratch_types=[
            pltpu.VMEM((LANES,), jnp.int32),
            pltpu.VMEM((COLS,), jnp.float32),
            pltpu.VMEM((LANES,), jnp.float32),
        ],
    )
    return f(xf, s16)


def _combine_body(q_ref, e_ref, s_ref, o_ref):
    q = q_ref[...]
    ii = (lax.broadcasted_iota(jnp.int32, (4, 128), 0) * 128
          + lax.broadcasted_iota(jnp.int32, (4, 128), 1))
    res = jnp.float32(1.0)
    for i in range(NSEG):
        a = s_ref[0, i]
        b = s_ref[0, i + 1]
        lo = lax.div(a + BROW - 1, BROW)
        hi = lax.div(b, BROW)
        m = jnp.logical_and(ii >= lo, ii < hi)
        bs = jnp.sum(jnp.where(m, q, 0.0))
        es = jnp.sum(e_ref[i, :])
        res = res * (bs + es)
    o_ref[...] = jnp.broadcast_to(res, (1, 1))


@jax.jit
def _combine(q, e, s16):
    return pl.pallas_call(
        _combine_body,
        out_shape=jax.ShapeDtypeStruct((1, 1), jnp.float32),
    )(q.reshape(4, 128), e, s16.reshape(1, LANES))


def kernel(x, slices):
    s16 = jnp.pad(slices.astype(jnp.int32), (0, 7))
    e = _edges(x, s16)
    q = _blocksums(x)
    res = _combine(q, e, s16)
    return res[0, 0]


# speedup vs baseline: 1.4841x; 1.1977x over previous
import jax
import jax.numpy as jnp
from jax import lax
from jax.experimental import pallas as pl
from jax.experimental.pallas import tpu as pltpu
from jax.experimental.pallas import tpu_sc as plsc

ROWS = 8192
COLS = 512
LANES = 16
NSEG = 8
RBLK = 1024


def _rowsum_body(x_ref, o_ref):
    o_ref[...] = jnp.sum(x_ref[...], axis=1)


@jax.jit
def _rowsums(x):
    return pl.pallas_call(
        _rowsum_body,
        grid=(ROWS // RBLK,),
        in_specs=[pl.BlockSpec((RBLK, COLS), lambda i: (i, 0))],
        out_specs=pl.BlockSpec((RBLK,), lambda i: (i,)),
        out_shape=jax.ShapeDtypeStruct((ROWS,), jnp.float32),
    )(x)


def _segprod_body(r_hbm, s_hbm, o_hbm, rbuf, sbuf, obuf):
    cid = lax.axis_index("c")
    sid = lax.axis_index("s")

    @pl.when(jnp.logical_and(cid == 0, sid == 0))
    def _():
        pltpu.sync_copy(s_hbm, sbuf)
        pltpu.sync_copy(r_hbm, rbuf)
        svec = sbuf[...]
        lane = lax.iota(jnp.int32, 16)
        zero = jnp.zeros((LANES,), jnp.float32)
        res = jnp.float32(1.0)
        for i in range(NSEG):
            a = svec[i]
            b = svec[i + 1]
            v0 = lax.div(a, LANES)
            v1 = lax.div(b + (LANES - 1), LANES)

            def body(v, acc, a=a, b=b):
                base = v * LANES
                vec = rbuf[pl.ds(base, LANES)]
                idx = base + lane
                m = (idx >= a) & (idx < b)
                return acc + jnp.where(m, vec, 0.0)

            acc = lax.fori_loop(v0, v1, body, zero)
            p = [acc[j] for j in range(LANES)]
            while len(p) > 1:
                p = [p[j] + p[j + 1] for j in range(0, len(p), 2)]
            res = res * p[0]
        obuf[...] = jnp.broadcast_to(res, (LANES,))
        pltpu.sync_copy(obuf, o_hbm)


@jax.jit
def _segprod(rowsums, s16):
    mesh = plsc.VectorSubcoreMesh(
        core_axis_name="c", subcore_axis_name="s", num_cores=2,
        num_subcores=16)
    f = pl.kernel(
        _segprod_body,
        out_type=jax.ShapeDtypeStruct((LANES,), jnp.float32),
        mesh=mesh,
        scratch_types=[
            pltpu.VMEM((ROWS,), jnp.float32),
            pltpu.VMEM((LANES,), jnp.int32),
            pltpu.VMEM((LANES,), jnp.float32),
        ],
    )
    return f(rowsums, s16)


def kernel(x, slices):
    s16 = jnp.pad(slices.astype(jnp.int32), (0, 7))
    rowsums = _rowsums(x)
    out = _segprod(rowsums, s16)
    return out[0]


# speedup vs baseline: 1.5772x; 1.0627x over previous
import jax
import jax.numpy as jnp
from jax import lax
from jax.experimental import pallas as pl
from jax.experimental.pallas import tpu as pltpu
from jax.experimental.pallas import tpu_sc as plsc

ROWS = 8192
COLS = 512
LANES = 16
NSEG = 8
RBLK = 1024


def _rowsum_body(x_ref, o_ref):
    o_ref[...] = jnp.sum(x_ref[...], axis=1)


@jax.jit
def _rowsums(x):
    return pl.pallas_call(
        _rowsum_body,
        grid=(ROWS // RBLK,),
        in_specs=[pl.BlockSpec((RBLK, COLS), lambda i: (i, 0))],
        out_specs=pl.BlockSpec((RBLK,), lambda i: (i,)),
        out_shape=jax.ShapeDtypeStruct((ROWS,), jnp.float32),
    )(x)


def _segprod_body(r_hbm, s_hbm, o_hbm, rbuf, sbuf, obuf):
    cid = lax.axis_index("c")
    sid = lax.axis_index("s")

    @pl.when(jnp.logical_and(cid == 0, sid == 0))
    def _():
        pltpu.sync_copy(s_hbm, sbuf)
        pltpu.sync_copy(r_hbm, rbuf)
        svec = sbuf[...]
        lane = lax.iota(jnp.int32, 16)
        zero = jnp.zeros((LANES,), jnp.float32)
        res = jnp.float32(1.0)
        for i in range(NSEG):
            a = svec[i]
            b = svec[i + 1]
            v0 = lax.div(a, LANES)
            v1 = lax.div(b + (LANES - 1), LANES)

            def body(v, acc, a=a, b=b):
                base = v * LANES
                vec = rbuf[pl.ds(base, LANES)]
                idx = base + lane
                m = (idx >= a) & (idx < b)
                return acc + jnp.where(m, vec, 0.0)

            acc = lax.fori_loop(v0, v1, body, zero)
            p = [acc[j] for j in range(LANES)]
            while len(p) > 1:
                p = [p[j] + p[j + 1] for j in range(0, len(p), 2)]
            res = res * p[0]
        obuf[...] = jnp.broadcast_to(res, (LANES,))
        pltpu.sync_copy(obuf, o_hbm)


@jax.jit
def _segprod(rowsums, s16):
    mesh = plsc.VectorSubcoreMesh(
        core_axis_name="c", subcore_axis_name="s", num_cores=1,
        num_subcores=16)
    f = pl.kernel(
        _segprod_body,
        out_type=jax.ShapeDtypeStruct((LANES,), jnp.float32),
        mesh=mesh,
        scratch_types=[
            pltpu.VMEM((ROWS,), jnp.float32),
            pltpu.VMEM((LANES,), jnp.int32),
            pltpu.VMEM((LANES,), jnp.float32),
        ],
    )
    return f(rowsums, s16)


def kernel(x, slices):
    s16 = jnp.pad(slices.astype(jnp.int32), (0, 7))
    rowsums = _rowsums(x)
    out = _segprod(rowsums, s16)
    return out[0]
